# baseline (speedup 1.0000x reference)
import functools

import jax
import jax.numpy as jnp
from jax import lax
from jax.experimental import pallas as pl
from jax.experimental.pallas import tpu as pltpu
from jax.experimental.pallas import tpu_sc as plsc

BATCH = 16384
N_FIELDS = 26
EMBED_DIM = 16
VOCAB = 1000000

B = BATCH * N_FIELDS
NC, NS = 2, 16
NW = NC * NS
C_SB = 512
N_SB_W = 61
SB_COVER = NW * N_SB_W * C_SB
N_LEFT = (VOCAB - SB_COVER) // 128
TAIL_START = SB_COVER + N_LEFT * 128
TAIL_N = VOCAB - TAIL_START
NBLOCKS = N_FIELDS * (BATCH // 128)
BLK_PER_W = NBLOCKS // NW
CHUNK_BLKS = 8
NCHUNK = BLK_PER_W // CHUNK_BLKS

_mesh = plsc.VectorSubcoreMesh(core_axis_name="c", subcore_axis_name="s")

_detile_params = pltpu.CompilerParams(use_tc_tiling_on_sc=True,
                                      needs_layout_passes=False)
_gather_params = pltpu.CompilerParams(use_tc_tiling_on_sc=False,
                                      needs_layout_passes=False)


def _iota16():
    return lax.iota(jnp.int32, 16)


@functools.partial(
    pl.kernel,
    mesh=_mesh,
    out_type=(
        jax.ShapeDtypeStruct((VOCAB * EMBED_DIM,), jnp.float32),
        jax.ShapeDtypeStruct((B,), jnp.int32),
    ),
    compiler_params=_detile_params,
    scratch_types=[
        pltpu.VMEM((EMBED_DIM, C_SB), jnp.float32),
        pltpu.VMEM((EMBED_DIM, C_SB), jnp.float32),
        pltpu.VMEM((C_SB * EMBED_DIM,), jnp.float32),
        pltpu.VMEM((C_SB * EMBED_DIM,), jnp.float32),
        pltpu.VMEM((BATCH,), jnp.int32),
        pltpu.VMEM((EMBED_DIM, TAIL_N), jnp.float32),
        pltpu.VMEM((TAIL_N * EMBED_DIM,), jnp.float32),
        pltpu.SemaphoreType.DMA,
        pltpu.SemaphoreType.DMA,
        pltpu.SemaphoreType.DMA,
        pltpu.SemaphoreType.DMA,
    ],
)
def _detile(tt, it, tab_lin, idx_lin, vin0, vin1, vout0, vout1, idxrow,
            tailbuf, tailout, isem0, isem1, osem0, osem1):
    wid = lax.axis_index("s") * NC + lax.axis_index("c")
    i16 = _iota16()
    vin_ = (vin0, vin1)
    vout_ = (vout0, vout1)
    isem_ = (isem0, isem1)
    osem_ = (osem0, osem1)

    @pl.when(wid < N_FIELDS)
    def _():
        pltpu.sync_copy(it.at[wid], idxrow)
        pltpu.sync_copy(idxrow, idx_lin.at[pl.ds(wid * BATCH, BATCH)])

    base_col = wid * N_SB_W * C_SB

    def in_desc(t, p):
        return pltpu.make_async_copy(
            tt.at[:, pl.ds(base_col + t * C_SB, C_SB)], vin_[p], isem_[p])

    def out_desc(t, p):
        return pltpu.make_async_copy(
            vout_[p],
            tab_lin.at[pl.ds((base_col + t * C_SB) * EMBED_DIM,
                             C_SB * EMBED_DIM)],
            osem_[p])

    i16x16 = i16 * EMBED_DIM

    def transpose_sb(p):
        for e in range(EMBED_DIM):
            def c_body(q, idxv, e=e):
                for u in range(4):
                    v = vin_[p][e, pl.ds((q * 4 + u) * 16, 16)]
                    plsc.store_scatter(vout_[p], [idxv + (256 * u)], v)
                return idxv + 1024

            lax.fori_loop(0, C_SB // 64, c_body, i16x16 + e)

    in_desc(0, 0).start()
    in_desc(0, 0).wait()
    in_desc(1, 1).start()
    transpose_sb(0)
    out_desc(0, 0).start()
    in_desc(2, 0).start()
    in_desc(1, 1).wait()
    transpose_sb(1)
    out_desc(1, 1).start()
    in_desc(3, 1).start()

    def pair_body(u, c):
        t0 = 2 * u
        t1 = t0 + 1
        in_desc(t0, 0).wait()
        out_desc(t0 - 2, 0).wait()
        transpose_sb(0)
        out_desc(t0, 0).start()
        in_desc(t0 + 2, 0).start()
        in_desc(t1, 1).wait()
        out_desc(t1 - 2, 1).wait()
        transpose_sb(1)
        out_desc(t1, 1).start()

        @pl.when(t1 + 2 < N_SB_W)
        def _():
            in_desc(t1 + 2, 1).start()

        return c

    lax.fori_loop(1, (N_SB_W - 1) // 2, pair_body, 0)
    in_desc(N_SB_W - 1, 0).wait()
    out_desc(N_SB_W - 3, 0).wait()
    transpose_sb(0)
    out_desc(N_SB_W - 1, 0).start()
    out_desc(N_SB_W - 2, 1).wait()
    out_desc(N_SB_W - 1, 0).wait()

    @pl.when((wid >= 16) & (wid < 16 + N_LEFT))
    def _():
        c0 = SB_COVER + (wid - 16) * 128
        pltpu.sync_copy(tt.at[:, pl.ds(c0, 128)], vin_[0].at[:, pl.ds(0, 128)])

        def trl_body(m, c2):
            for jj in range(8):
                r = m * 8 + jj
                v = plsc.load_gather(vin_[0],
                                     [i16, jnp.full((16,), r, jnp.int32)])
                vout_[0][pl.ds(r * EMBED_DIM, EMBED_DIM)] = v
            return c2

        lax.fori_loop(0, 16, trl_body, 0)
        pltpu.sync_copy(vout_[0].at[pl.ds(0, 128 * EMBED_DIM)],
                        tab_lin.at[pl.ds(c0 * EMBED_DIM, 128 * EMBED_DIM)])

    @pl.when(wid == 20)
    def _():
        def te_body(e, c2):
            pltpu.sync_copy(tt.at[e, pl.ds(TAIL_START, TAIL_N)], tailbuf.at[e])
            return c2

        lax.fori_loop(0, EMBED_DIM, te_body, 0)

        def tr2_body(r, c2):
            v = plsc.load_gather(tailbuf, [i16, jnp.full((16,), r, jnp.int32)])
            tailout[pl.ds(r * EMBED_DIM, EMBED_DIM)] = v
            return c2

        lax.fori_loop(0, TAIL_N, tr2_body, 0)
        pltpu.sync_copy(tailout,
                        tab_lin.at[pl.ds(TAIL_START * EMBED_DIM,
                                         TAIL_N * EMBED_DIM)])


@functools.partial(
    pl.kernel,
    mesh=_mesh,
    out_type=jax.ShapeDtypeStruct((N_FIELDS, 2, BATCH // 128, 8, 128),
                                  jnp.float32),
    compiler_params=_gather_params,
    scratch_types=[
        pltpu.VMEM((BLK_PER_W * 128,), jnp.int32),
        pltpu.VMEM((CHUNK_BLKS * 128, EMBED_DIM), jnp.float32),
        pltpu.VMEM((CHUNK_BLKS * 128, EMBED_DIM), jnp.float32),
        pltpu.VMEM((2, CHUNK_BLKS, 8, 128), jnp.float32),
        pltpu.VMEM((2, CHUNK_BLKS, 8, 128), jnp.float32),
        pltpu.SemaphoreType.DMA,
        pltpu.SemaphoreType.DMA,
        pltpu.SemaphoreType.DMA,
        pltpu.SemaphoreType.DMA,
    ],
)
def _gather(tab2d, idx_lin, out5, idxall, rows0, rows1, outb0, outb1,
            gsem0, gsem1, wsem0, wsem1):
    wid = lax.axis_index("s") * NC + lax.axis_index("c")
    i16 = _iota16()
    rows_ = (rows0, rows1)
    outb_ = (outb0, outb1)
    gsem_ = (gsem0, gsem1)
    wsem_ = (wsem0, wsem1)
    nbt = BATCH // 128

    base_g = wid * BLK_PER_W
    pltpu.sync_copy(idx_lin.at[pl.ds(base_g * 128, BLK_PER_W * 128)], idxall)

    def g_desc(j, p):
        return pltpu.make_async_copy(
            tab2d.at[idxall.at[pl.ds(j * CHUNK_BLKS * 128, CHUNK_BLKS * 128)]],
            rows_[p], gsem_[p])

    def out_descs(j, p):
        g0 = base_g + CHUNK_BLKS * j
        f = g0 // nbt
        bt0 = g0 % nbt
        return (
            pltpu.make_async_copy(
                outb_[p].at[0], out5.at[f, 0, pl.ds(bt0, CHUNK_BLKS)],
                wsem_[p]),
            pltpu.make_async_copy(
                outb_[p].at[1], out5.at[f, 1, pl.ds(bt0, CHUNK_BLKS)],
                wsem_[p]),
        )

    def transpose_chunk(p):
        def e_body(e, c2, p=p):
            g2 = e // 8
            e8 = e % 8
            cv = jnp.full((16,), e, jnp.int32)

            def k_body(k, rv, p=p):
                for j2 in range(8):
                    v = plsc.load_gather(rows_[p], [rv, cv])
                    outb_[p][g2, k, e8, pl.ds(16 * j2, 16)] = v
                    rv = rv + 16
                return rv

            lax.fori_loop(0, CHUNK_BLKS, k_body, i16)
            return c2

        lax.fori_loop(0, EMBED_DIM, e_body, 0)

    def do_chunk(j, p):
        transpose_chunk(p)
        d1, d2 = out_descs(j, p)
        d1.start()
        d2.start()

    def wait_outs(j, p):
        d1, d2 = out_descs(j, p)
        d1.wait()
        d2.wait()

    g_desc(0, 0).start()
    g_desc(1, 1).start()
    g_desc(0, 0).wait()
    do_chunk(0, 0)
    g_desc(2, 0).start()
    g_desc(1, 1).wait()
    do_chunk(1, 1)
    g_desc(3, 1).start()

    def pair_body(u, c):
        j0 = 2 * u
        j1 = j0 + 1
        g_desc(j0, 0).wait()
        wait_outs(j0 - 2, 0)
        do_chunk(j0, 0)
        g_desc(j0 + 2, 0).start()
        g_desc(j1, 1).wait()
        wait_outs(j1 - 2, 1)
        do_chunk(j1, 1)

        @pl.when(j1 + 2 < NCHUNK)
        def _():
            g_desc(j1 + 2, 1).start()

        return c

    lax.fori_loop(1, (NCHUNK - 1) // 2, pair_body, 0)
    g_desc(NCHUNK - 1, 0).wait()
    wait_outs(NCHUNK - 3, 0)
    do_chunk(NCHUNK - 1, 0)
    wait_outs(NCHUNK - 2, 1)
    wait_outs(NCHUNK - 1, 0)


@jax.jit
def kernel(indices, table):
    tab_lin, idx_lin = _detile(table.T, indices.T)
    out5 = _gather(tab_lin.reshape(VOCAB, EMBED_DIM), idx_lin)
    return out5.transpose(2, 4, 0, 1, 3).reshape(BATCH, N_FIELDS, EMBED_DIM)


# speedup vs baseline: 1.1059x; 1.1059x over previous
import functools

import jax
import jax.numpy as jnp
from jax import lax
from jax.experimental import pallas as pl
from jax.experimental.pallas import tpu as pltpu
from jax.experimental.pallas import tpu_sc as plsc

BATCH = 16384
N_FIELDS = 26
EMBED_DIM = 16
VOCAB = 1000000

B = BATCH * N_FIELDS
NC, NS = 2, 16
NW = NC * NS
C_SB = 512
N_SB_W = 61
SB_COVER = NW * N_SB_W * C_SB
N_LEFT = (VOCAB - SB_COVER) // 128
TAIL_START = SB_COVER + N_LEFT * 128
TAIL_N = VOCAB - TAIL_START
NBLOCKS = N_FIELDS * (BATCH // 128)
BLK_PER_W = NBLOCKS // NW
CHUNK_BLKS = 8
NCHUNK = BLK_PER_W // CHUNK_BLKS

_mesh = plsc.VectorSubcoreMesh(core_axis_name="c", subcore_axis_name="s")

_detile_params = pltpu.CompilerParams(use_tc_tiling_on_sc=True,
                                      needs_layout_passes=False)
_gather_params = pltpu.CompilerParams(use_tc_tiling_on_sc=False,
                                      needs_layout_passes=False)


def _iota16():
    return lax.iota(jnp.int32, 16)


@functools.partial(
    pl.kernel,
    mesh=_mesh,
    out_type=(
        jax.ShapeDtypeStruct((VOCAB * EMBED_DIM,), jnp.float32),
        jax.ShapeDtypeStruct((B,), jnp.int32),
    ),
    compiler_params=_detile_params,
    scratch_types=[
        pltpu.VMEM((EMBED_DIM, C_SB), jnp.float32),
        pltpu.VMEM((EMBED_DIM, C_SB), jnp.float32),
        pltpu.VMEM((C_SB * EMBED_DIM,), jnp.float32),
        pltpu.VMEM((C_SB * EMBED_DIM,), jnp.float32),
        pltpu.VMEM((BATCH,), jnp.int32),
        pltpu.VMEM((EMBED_DIM, TAIL_N), jnp.float32),
        pltpu.VMEM((TAIL_N * EMBED_DIM,), jnp.float32),
        pltpu.SemaphoreType.DMA,
        pltpu.SemaphoreType.DMA,
        pltpu.SemaphoreType.DMA,
        pltpu.SemaphoreType.DMA,
    ],
)
def _detile(tt, it, tab_lin, idx_lin, vin0, vin1, vout0, vout1, idxrow,
            tailbuf, tailout, isem0, isem1, osem0, osem1):
    wid = lax.axis_index("s") * NC + lax.axis_index("c")
    i16 = _iota16()
    vin_ = (vin0, vin1)
    vout_ = (vout0, vout1)
    isem_ = (isem0, isem1)
    osem_ = (osem0, osem1)

    @pl.when(wid < N_FIELDS)
    def _():
        pltpu.sync_copy(it.at[wid], idxrow)
        pltpu.sync_copy(idxrow, idx_lin.at[pl.ds(wid * BATCH, BATCH)])

    base_col = wid * N_SB_W * C_SB

    def in_desc(t, p):
        return pltpu.make_async_copy(
            tt.at[:, pl.ds(base_col + t * C_SB, C_SB)], vin_[p], isem_[p])

    def out_desc(t, p):
        return pltpu.make_async_copy(
            vout_[p],
            tab_lin.at[pl.ds((base_col + t * C_SB) * EMBED_DIM,
                             C_SB * EMBED_DIM)],
            osem_[p])

    i16x16 = i16 * EMBED_DIM

    def transpose_sb(p):
        def e_body(e, c2):
            def c_body(q, idxv):
                for u in range(4):
                    v = vin_[p][e, pl.ds((q * 4 + u) * 16, 16)]
                    plsc.store_scatter(vout_[p], [idxv + (256 * u)], v)
                return idxv + 1024

            lax.fori_loop(0, C_SB // 64, c_body, i16x16 + e)
            return c2

        lax.fori_loop(0, EMBED_DIM, e_body, 0)

    in_desc(0, 0).start()
    in_desc(0, 0).wait()
    in_desc(1, 1).start()
    transpose_sb(0)
    out_desc(0, 0).start()
    in_desc(2, 0).start()
    in_desc(1, 1).wait()
    transpose_sb(1)
    out_desc(1, 1).start()
    in_desc(3, 1).start()

    def pair_body(u, c):
        t0 = 2 * u
        t1 = t0 + 1
        in_desc(t0, 0).wait()
        out_desc(t0 - 2, 0).wait()
        transpose_sb(0)
        out_desc(t0, 0).start()
        in_desc(t0 + 2, 0).start()
        in_desc(t1, 1).wait()
        out_desc(t1 - 2, 1).wait()
        transpose_sb(1)
        out_desc(t1, 1).start()

        @pl.when(t1 + 2 < N_SB_W)
        def _():
            in_desc(t1 + 2, 1).start()

        return c

    lax.fori_loop(1, (N_SB_W - 1) // 2, pair_body, 0)
    in_desc(N_SB_W - 1, 0).wait()
    out_desc(N_SB_W - 3, 0).wait()
    transpose_sb(0)
    out_desc(N_SB_W - 1, 0).start()
    out_desc(N_SB_W - 2, 1).wait()
    out_desc(N_SB_W - 1, 0).wait()

    @pl.when((wid >= 16) & (wid < 16 + N_LEFT))
    def _():
        c0 = SB_COVER + (wid - 16) * 128
        pltpu.sync_copy(tt.at[:, pl.ds(c0, 128)], vin_[0].at[:, pl.ds(0, 128)])

        def trl_body(m, c2):
            for jj in range(8):
                r = m * 8 + jj
                v = plsc.load_gather(vin_[0],
                                     [i16, jnp.full((16,), r, jnp.int32)])
                vout_[0][pl.ds(r * EMBED_DIM, EMBED_DIM)] = v
            return c2

        lax.fori_loop(0, 16, trl_body, 0)
        pltpu.sync_copy(vout_[0].at[pl.ds(0, 128 * EMBED_DIM)],
                        tab_lin.at[pl.ds(c0 * EMBED_DIM, 128 * EMBED_DIM)])

    @pl.when(wid == 20)
    def _():
        def te_body(e, c2):
            pltpu.sync_copy(tt.at[e, pl.ds(TAIL_START, TAIL_N)], tailbuf.at[e])
            return c2

        lax.fori_loop(0, EMBED_DIM, te_body, 0)

        def tr2_body(r, c2):
            v = plsc.load_gather(tailbuf, [i16, jnp.full((16,), r, jnp.int32)])
            tailout[pl.ds(r * EMBED_DIM, EMBED_DIM)] = v
            return c2

        lax.fori_loop(0, TAIL_N, tr2_body, 0)
        pltpu.sync_copy(tailout,
                        tab_lin.at[pl.ds(TAIL_START * EMBED_DIM,
                                         TAIL_N * EMBED_DIM)])


@functools.partial(
    pl.kernel,
    mesh=_mesh,
    out_type=jax.ShapeDtypeStruct((N_FIELDS, 2, BATCH // 128, 8, 128),
                                  jnp.float32),
    compiler_params=_gather_params,
    scratch_types=[
        pltpu.VMEM((BLK_PER_W * 128,), jnp.int32),
        pltpu.VMEM((CHUNK_BLKS * 128, EMBED_DIM), jnp.float32),
        pltpu.VMEM((CHUNK_BLKS * 128, EMBED_DIM), jnp.float32),
        pltpu.VMEM((2, CHUNK_BLKS, 8, 128), jnp.float32),
        pltpu.VMEM((2, CHUNK_BLKS, 8, 128), jnp.float32),
        pltpu.SemaphoreType.DMA,
        pltpu.SemaphoreType.DMA,
        pltpu.SemaphoreType.DMA,
        pltpu.SemaphoreType.DMA,
    ],
)
def _gather(tab2d, idx_lin, out5, idxall, rows0, rows1, outb0, outb1,
            gsem0, gsem1, wsem0, wsem1):
    wid = lax.axis_index("s") * NC + lax.axis_index("c")
    i16 = _iota16()
    rows_ = (rows0, rows1)
    outb_ = (outb0, outb1)
    gsem_ = (gsem0, gsem1)
    wsem_ = (wsem0, wsem1)
    nbt = BATCH // 128

    base_g = wid * BLK_PER_W
    pltpu.sync_copy(idx_lin.at[pl.ds(base_g * 128, BLK_PER_W * 128)], idxall)

    def g_desc(j, p):
        return pltpu.make_async_copy(
            tab2d.at[idxall.at[pl.ds(j * CHUNK_BLKS * 128, CHUNK_BLKS * 128)]],
            rows_[p], gsem_[p])

    def out_descs(j, p):
        g0 = base_g + CHUNK_BLKS * j
        f = g0 // nbt
        bt0 = g0 % nbt
        return (
            pltpu.make_async_copy(
                outb_[p].at[0], out5.at[f, 0, pl.ds(bt0, CHUNK_BLKS)],
                wsem_[p]),
            pltpu.make_async_copy(
                outb_[p].at[1], out5.at[f, 1, pl.ds(bt0, CHUNK_BLKS)],
                wsem_[p]),
        )

    def transpose_chunk(p):
        def e_body(e, c2, p=p):
            g2 = e // 8
            e8 = e % 8
            cv = jnp.full((16,), e, jnp.int32)

            def k_body(k, rv0, p=p):
                for j2 in range(8):
                    v = plsc.load_gather(rows_[p], [rv0 + 16 * j2, cv])
                    outb_[p][g2, k, e8, pl.ds(16 * j2, 16)] = v
                return rv0 + 128

            lax.fori_loop(0, CHUNK_BLKS, k_body, i16)
            return c2

        lax.fori_loop(0, EMBED_DIM, e_body, 0)

    def do_chunk(j, p):
        transpose_chunk(p)
        d1, d2 = out_descs(j, p)
        d1.start()
        d2.start()

    def wait_outs(j, p):
        d1, d2 = out_descs(j, p)
        d1.wait()
        d2.wait()

    g_desc(0, 0).start()
    g_desc(1, 1).start()
    g_desc(0, 0).wait()
    do_chunk(0, 0)
    g_desc(2, 0).start()
    g_desc(1, 1).wait()
    do_chunk(1, 1)
    g_desc(3, 1).start()

    def pair_body(u, c):
        j0 = 2 * u
        j1 = j0 + 1
        g_desc(j0, 0).wait()
        wait_outs(j0 - 2, 0)
        do_chunk(j0, 0)
        g_desc(j0 + 2, 0).start()
        g_desc(j1, 1).wait()
        wait_outs(j1 - 2, 1)
        do_chunk(j1, 1)

        @pl.when(j1 + 2 < NCHUNK)
        def _():
            g_desc(j1 + 2, 1).start()

        return c

    lax.fori_loop(1, (NCHUNK - 1) // 2, pair_body, 0)
    g_desc(NCHUNK - 1, 0).wait()
    wait_outs(NCHUNK - 3, 0)
    do_chunk(NCHUNK - 1, 0)
    wait_outs(NCHUNK - 2, 1)
    wait_outs(NCHUNK - 1, 0)


@jax.jit
def kernel(indices, table):
    tab_lin, idx_lin = _detile(table.T, indices.T)
    out5 = _gather(tab_lin.reshape(VOCAB, EMBED_DIM), idx_lin)
    return out5.transpose(2, 4, 0, 1, 3).reshape(BATCH, N_FIELDS, EMBED_DIM)


# speedup vs baseline: 1.1072x; 1.0012x over previous
import functools

import jax
import jax.numpy as jnp
from jax import lax
from jax.experimental import pallas as pl
from jax.experimental.pallas import tpu as pltpu
from jax.experimental.pallas import tpu_sc as plsc

BATCH = 16384
N_FIELDS = 26
EMBED_DIM = 16
VOCAB = 1000000

B = BATCH * N_FIELDS
NC, NS = 2, 16
NW = NC * NS
C_SB = 512
N_SB_W = 61
SB_COVER = NW * N_SB_W * C_SB
N_LEFT = (VOCAB - SB_COVER) // 128
TAIL_START = SB_COVER + N_LEFT * 128
TAIL_N = VOCAB - TAIL_START
NBLOCKS = N_FIELDS * (BATCH // 128)
BLK_PER_W = NBLOCKS // NW
CHUNK_BLKS = 8
NCHUNK = BLK_PER_W // CHUNK_BLKS

_mesh = plsc.VectorSubcoreMesh(core_axis_name="c", subcore_axis_name="s")

_detile_params = pltpu.CompilerParams(use_tc_tiling_on_sc=True,
                                      needs_layout_passes=False)
_gather_params = pltpu.CompilerParams(use_tc_tiling_on_sc=False,
                                      needs_layout_passes=False)


def _iota16():
    return lax.iota(jnp.int32, 16)


@functools.partial(
    pl.kernel,
    mesh=_mesh,
    out_type=(
        jax.ShapeDtypeStruct((VOCAB * EMBED_DIM,), jnp.float32),
        jax.ShapeDtypeStruct((B,), jnp.int32),
    ),
    compiler_params=_detile_params,
    scratch_types=[
        pltpu.VMEM((EMBED_DIM, C_SB), jnp.float32),
        pltpu.VMEM((EMBED_DIM, C_SB), jnp.float32),
        pltpu.VMEM((C_SB * EMBED_DIM,), jnp.float32),
        pltpu.VMEM((C_SB * EMBED_DIM,), jnp.float32),
        pltpu.VMEM((BATCH,), jnp.int32),
        pltpu.VMEM((EMBED_DIM, TAIL_N), jnp.float32),
        pltpu.VMEM((TAIL_N * EMBED_DIM,), jnp.float32),
        pltpu.SemaphoreType.DMA,
        pltpu.SemaphoreType.DMA,
        pltpu.SemaphoreType.DMA,
        pltpu.SemaphoreType.DMA,
    ],
)
def _detile(tt, it, tab_lin, idx_lin, vin0, vin1, vout0, vout1, idxrow,
            tailbuf, tailout, isem0, isem1, osem0, osem1):
    wid = lax.axis_index("s") * NC + lax.axis_index("c")
    i16 = _iota16()
    vin_ = (vin0, vin1)
    vout_ = (vout0, vout1)
    isem_ = (isem0, isem1)
    osem_ = (osem0, osem1)

    @pl.when(wid < N_FIELDS)
    def _():
        pltpu.sync_copy(it.at[wid], idxrow)
        pltpu.sync_copy(idxrow, idx_lin.at[pl.ds(wid * BATCH, BATCH)])

    base_col = wid * N_SB_W * C_SB

    def in_desc(t, p):
        return pltpu.make_async_copy(
            tt.at[:, pl.ds(base_col + t * C_SB, C_SB)], vin_[p], isem_[p])

    def out_desc(t, p):
        return pltpu.make_async_copy(
            vout_[p],
            tab_lin.at[pl.ds((base_col + t * C_SB) * EMBED_DIM,
                             C_SB * EMBED_DIM)],
            osem_[p])

    i16x16 = i16 * EMBED_DIM

    def transpose_sb(p):
        def e_body(e, c2):
            def c_body(q, idxv):
                for u in range(8):
                    v = vin_[p][e, pl.ds((q * 8 + u) * 16, 16)]
                    plsc.store_scatter(vout_[p], [idxv + (256 * u)], v)
                return idxv + 2048

            lax.fori_loop(0, C_SB // 128, c_body, i16x16 + e)
            return c2

        lax.fori_loop(0, EMBED_DIM, e_body, 0)

    in_desc(0, 0).start()
    in_desc(0, 0).wait()
    in_desc(1, 1).start()
    transpose_sb(0)
    out_desc(0, 0).start()
    in_desc(2, 0).start()
    in_desc(1, 1).wait()
    transpose_sb(1)
    out_desc(1, 1).start()
    in_desc(3, 1).start()

    def pair_body(u, c):
        t0 = 2 * u
        t1 = t0 + 1
        in_desc(t0, 0).wait()
        out_desc(t0 - 2, 0).wait()
        transpose_sb(0)
        out_desc(t0, 0).start()
        in_desc(t0 + 2, 0).start()
        in_desc(t1, 1).wait()
        out_desc(t1 - 2, 1).wait()
        transpose_sb(1)
        out_desc(t1, 1).start()

        @pl.when(t1 + 2 < N_SB_W)
        def _():
            in_desc(t1 + 2, 1).start()

        return c

    lax.fori_loop(1, (N_SB_W - 1) // 2, pair_body, 0)
    in_desc(N_SB_W - 1, 0).wait()
    out_desc(N_SB_W - 3, 0).wait()
    transpose_sb(0)
    out_desc(N_SB_W - 1, 0).start()
    out_desc(N_SB_W - 2, 1).wait()
    out_desc(N_SB_W - 1, 0).wait()

    @pl.when((wid >= 16) & (wid < 16 + N_LEFT))
    def _():
        c0 = SB_COVER + (wid - 16) * 128
        pltpu.sync_copy(tt.at[:, pl.ds(c0, 128)], vin_[0].at[:, pl.ds(0, 128)])

        def trl_body(m, c2):
            for jj in range(8):
                r = m * 8 + jj
                v = plsc.load_gather(vin_[0],
                                     [i16, jnp.full((16,), r, jnp.int32)])
                vout_[0][pl.ds(r * EMBED_DIM, EMBED_DIM)] = v
            return c2

        lax.fori_loop(0, 16, trl_body, 0)
        pltpu.sync_copy(vout_[0].at[pl.ds(0, 128 * EMBED_DIM)],
                        tab_lin.at[pl.ds(c0 * EMBED_DIM, 128 * EMBED_DIM)])

    @pl.when(wid == 20)
    def _():
        def te_body(e, c2):
            pltpu.sync_copy(tt.at[e, pl.ds(TAIL_START, TAIL_N)], tailbuf.at[e])
            return c2

        lax.fori_loop(0, EMBED_DIM, te_body, 0)

        def tr2_body(r, c2):
            v = plsc.load_gather(tailbuf, [i16, jnp.full((16,), r, jnp.int32)])
            tailout[pl.ds(r * EMBED_DIM, EMBED_DIM)] = v
            return c2

        lax.fori_loop(0, TAIL_N, tr2_body, 0)
        pltpu.sync_copy(tailout,
                        tab_lin.at[pl.ds(TAIL_START * EMBED_DIM,
                                         TAIL_N * EMBED_DIM)])


@functools.partial(
    pl.kernel,
    mesh=_mesh,
    out_type=jax.ShapeDtypeStruct((N_FIELDS, 2, BATCH // 128, 8, 128),
                                  jnp.float32),
    compiler_params=_gather_params,
    scratch_types=[
        pltpu.VMEM((BLK_PER_W * 128,), jnp.int32),
        pltpu.VMEM((CHUNK_BLKS * 128, EMBED_DIM), jnp.float32),
        pltpu.VMEM((CHUNK_BLKS * 128, EMBED_DIM), jnp.float32),
        pltpu.VMEM((2, CHUNK_BLKS, 8, 128), jnp.float32),
        pltpu.VMEM((2, CHUNK_BLKS, 8, 128), jnp.float32),
        pltpu.SemaphoreType.DMA,
        pltpu.SemaphoreType.DMA,
        pltpu.SemaphoreType.DMA,
        pltpu.SemaphoreType.DMA,
    ],
)
def _gather(tab2d, idx_lin, out5, idxall, rows0, rows1, outb0, outb1,
            gsem0, gsem1, wsem0, wsem1):
    wid = lax.axis_index("s") * NC + lax.axis_index("c")
    i16 = _iota16()
    rows_ = (rows0, rows1)
    outb_ = (outb0, outb1)
    gsem_ = (gsem0, gsem1)
    wsem_ = (wsem0, wsem1)
    nbt = BATCH // 128

    base_g = wid * BLK_PER_W
    pltpu.sync_copy(idx_lin.at[pl.ds(base_g * 128, BLK_PER_W * 128)], idxall)

    def g_desc(j, p):
        return pltpu.make_async_copy(
            tab2d.at[idxall.at[pl.ds(j * CHUNK_BLKS * 128, CHUNK_BLKS * 128)]],
            rows_[p], gsem_[p])

    def out_descs(j, p):
        g0 = base_g + CHUNK_BLKS * j
        f = g0 // nbt
        bt0 = g0 % nbt
        return (
            pltpu.make_async_copy(
                outb_[p].at[0], out5.at[f, 0, pl.ds(bt0, CHUNK_BLKS)],
                wsem_[p]),
            pltpu.make_async_copy(
                outb_[p].at[1], out5.at[f, 1, pl.ds(bt0, CHUNK_BLKS)],
                wsem_[p]),
        )

    def transpose_chunk(p):
        def e_body(e, c2, p=p):
            g2 = e // 8
            e8 = e % 8
            cv = jnp.full((16,), e, jnp.int32)

            def k_body(k, rv0, p=p):
                for j2 in range(8):
                    v = plsc.load_gather(rows_[p], [rv0 + 16 * j2, cv])
                    outb_[p][g2, k, e8, pl.ds(16 * j2, 16)] = v
                return rv0 + 128

            lax.fori_loop(0, CHUNK_BLKS, k_body, i16)
            return c2

        lax.fori_loop(0, EMBED_DIM, e_body, 0)

    def do_chunk(j, p):
        transpose_chunk(p)
        d1, d2 = out_descs(j, p)
        d1.start()
        d2.start()

    def wait_outs(j, p):
        d1, d2 = out_descs(j, p)
        d1.wait()
        d2.wait()

    g_desc(0, 0).start()
    g_desc(1, 1).start()
    g_desc(0, 0).wait()
    do_chunk(0, 0)
    g_desc(2, 0).start()
    g_desc(1, 1).wait()
    do_chunk(1, 1)
    g_desc(3, 1).start()

    def pair_body(u, c):
        j0 = 2 * u
        j1 = j0 + 1
        g_desc(j0, 0).wait()
        wait_outs(j0 - 2, 0)
        do_chunk(j0, 0)
        g_desc(j0 + 2, 0).start()
        g_desc(j1, 1).wait()
        wait_outs(j1 - 2, 1)
        do_chunk(j1, 1)

        @pl.when(j1 + 2 < NCHUNK)
        def _():
            g_desc(j1 + 2, 1).start()

        return c

    lax.fori_loop(1, (NCHUNK - 1) // 2, pair_body, 0)
    g_desc(NCHUNK - 1, 0).wait()
    wait_outs(NCHUNK - 3, 0)
    do_chunk(NCHUNK - 1, 0)
    wait_outs(NCHUNK - 2, 1)
    wait_outs(NCHUNK - 1, 0)


@jax.jit
def kernel(indices, table):
    tab_lin, idx_lin = _detile(table.T, indices.T)
    out5 = _gather(tab_lin.reshape(VOCAB, EMBED_DIM), idx_lin)
    return out5.transpose(2, 4, 0, 1, 3).reshape(BATCH, N_FIELDS, EMBED_DIM)


# speedup vs baseline: 1.4638x; 1.3221x over previous
import functools

import jax
import jax.numpy as jnp
from jax import lax
from jax.experimental import pallas as pl
from jax.experimental.pallas import tpu as pltpu
from jax.experimental.pallas import tpu_sc as plsc

BATCH = 16384
N_FIELDS = 26
EMBED_DIM = 16
VOCAB = 1000000

B = BATCH * N_FIELDS
NC, NS = 2, 16
NW = NC * NS
C_SB = 512
N_SB_W = 61
SB_COVER = NW * N_SB_W * C_SB
N_LEFT = (VOCAB - SB_COVER) // 128
TAIL_START = SB_COVER + N_LEFT * 128
TAIL_N = VOCAB - TAIL_START
NBLOCKS = N_FIELDS * (BATCH // 128)
BLK_PER_W = NBLOCKS // NW
CHUNK_BLKS = 8
NCHUNK = BLK_PER_W // CHUNK_BLKS

_mesh = plsc.VectorSubcoreMesh(core_axis_name="c", subcore_axis_name="s")

_detile_params = pltpu.CompilerParams(use_tc_tiling_on_sc=True,
                                      needs_layout_passes=False)
_gather_params = pltpu.CompilerParams(use_tc_tiling_on_sc=False,
                                      needs_layout_passes=False)


def _iota16():
    return lax.iota(jnp.int32, 16)


@functools.partial(
    pl.kernel,
    mesh=_mesh,
    out_type=(
        jax.ShapeDtypeStruct((VOCAB * EMBED_DIM,), jnp.float32),
        jax.ShapeDtypeStruct((B,), jnp.int32),
    ),
    compiler_params=_detile_params,
    scratch_types=[
        pltpu.VMEM((EMBED_DIM, C_SB), jnp.float32),
        pltpu.VMEM((EMBED_DIM, C_SB), jnp.float32),
        pltpu.VMEM((C_SB * EMBED_DIM,), jnp.float32),
        pltpu.VMEM((C_SB * EMBED_DIM,), jnp.float32),
        pltpu.VMEM((BATCH,), jnp.int32),
        pltpu.VMEM((EMBED_DIM, TAIL_N), jnp.float32),
        pltpu.VMEM((TAIL_N * EMBED_DIM,), jnp.float32),
        pltpu.SemaphoreType.DMA,
        pltpu.SemaphoreType.DMA,
        pltpu.SemaphoreType.DMA,
        pltpu.SemaphoreType.DMA,
    ],
)
def _detile(tt, it, tab_lin, idx_lin, vin0, vin1, vout0, vout1, idxrow,
            tailbuf, tailout, isem0, isem1, osem0, osem1):
    wid = lax.axis_index("s") * NC + lax.axis_index("c")
    i16 = _iota16()
    vin_ = (vin0, vin1)
    vout_ = (vout0, vout1)
    isem_ = (isem0, isem1)
    osem_ = (osem0, osem1)

    @pl.when(wid < N_FIELDS)
    def _():
        pltpu.sync_copy(it.at[wid], idxrow)
        pltpu.sync_copy(idxrow, idx_lin.at[pl.ds(wid * BATCH, BATCH)])

    base_col = wid * N_SB_W * C_SB

    def in_desc(t, p):
        return pltpu.make_async_copy(
            tt.at[:, pl.ds(base_col + t * C_SB, C_SB)], vin_[p], isem_[p])

    def out_desc(t, p):
        return pltpu.make_async_copy(
            vout_[p],
            tab_lin.at[pl.ds((base_col + t * C_SB) * EMBED_DIM,
                             C_SB * EMBED_DIM)],
            osem_[p])

    diag = [(i16 + d) & 15 for d in range(16)]
    diag16 = [diag[d] * 16 + i16 for d in range(16)]

    def transpose_sb(p):
        def c_body(c0w, carry):
            c0 = c0w * 16
            for d in range(16):
                v = plsc.load_gather(vin_[p], [i16, diag[d] + c0])
                plsc.store_scatter(vout_[p], [diag16[d] + c0 * 16], v)
            return carry

        lax.fori_loop(0, C_SB // 16, c_body, 0)

    in_desc(0, 0).start()
    in_desc(0, 0).wait()
    in_desc(1, 1).start()
    transpose_sb(0)
    out_desc(0, 0).start()
    in_desc(2, 0).start()
    in_desc(1, 1).wait()
    transpose_sb(1)
    out_desc(1, 1).start()
    in_desc(3, 1).start()

    def pair_body(u, c):
        t0 = 2 * u
        t1 = t0 + 1
        in_desc(t0, 0).wait()
        out_desc(t0 - 2, 0).wait()
        transpose_sb(0)
        out_desc(t0, 0).start()
        in_desc(t0 + 2, 0).start()
        in_desc(t1, 1).wait()
        out_desc(t1 - 2, 1).wait()
        transpose_sb(1)
        out_desc(t1, 1).start()

        @pl.when(t1 + 2 < N_SB_W)
        def _():
            in_desc(t1 + 2, 1).start()

        return c

    lax.fori_loop(1, (N_SB_W - 1) // 2, pair_body, 0)
    in_desc(N_SB_W - 1, 0).wait()
    out_desc(N_SB_W - 3, 0).wait()
    transpose_sb(0)
    out_desc(N_SB_W - 1, 0).start()
    out_desc(N_SB_W - 2, 1).wait()
    out_desc(N_SB_W - 1, 0).wait()

    @pl.when((wid >= 16) & (wid < 16 + N_LEFT))
    def _():
        c0 = SB_COVER + (wid - 16) * 128
        pltpu.sync_copy(tt.at[:, pl.ds(c0, 128)], vin_[0].at[:, pl.ds(0, 128)])

        def trl_body(m, c2):
            for jj in range(8):
                r = m * 8 + jj
                v = plsc.load_gather(vin_[0],
                                     [i16, jnp.full((16,), r, jnp.int32)])
                vout_[0][pl.ds(r * EMBED_DIM, EMBED_DIM)] = v
            return c2

        lax.fori_loop(0, 16, trl_body, 0)
        pltpu.sync_copy(vout_[0].at[pl.ds(0, 128 * EMBED_DIM)],
                        tab_lin.at[pl.ds(c0 * EMBED_DIM, 128 * EMBED_DIM)])

    @pl.when(wid == 20)
    def _():
        def te_body(e, c2):
            pltpu.sync_copy(tt.at[e, pl.ds(TAIL_START, TAIL_N)], tailbuf.at[e])
            return c2

        lax.fori_loop(0, EMBED_DIM, te_body, 0)

        def tr2_body(r, c2):
            v = plsc.load_gather(tailbuf, [i16, jnp.full((16,), r, jnp.int32)])
            tailout[pl.ds(r * EMBED_DIM, EMBED_DIM)] = v
            return c2

        lax.fori_loop(0, TAIL_N, tr2_body, 0)
        pltpu.sync_copy(tailout,
                        tab_lin.at[pl.ds(TAIL_START * EMBED_DIM,
                                         TAIL_N * EMBED_DIM)])


@functools.partial(
    pl.kernel,
    mesh=_mesh,
    out_type=jax.ShapeDtypeStruct((N_FIELDS, 2, BATCH * 8), jnp.float32),
    compiler_params=_gather_params,
    scratch_types=[
        pltpu.VMEM((BLK_PER_W * 128,), jnp.int32),
        pltpu.VMEM((CHUNK_BLKS * 128, EMBED_DIM), jnp.float32),
        pltpu.VMEM((CHUNK_BLKS * 128, EMBED_DIM), jnp.float32),
        pltpu.VMEM((2 * CHUNK_BLKS * 8 * 128,), jnp.float32),
        pltpu.VMEM((2 * CHUNK_BLKS * 8 * 128,), jnp.float32),
        pltpu.SemaphoreType.DMA,
        pltpu.SemaphoreType.DMA,
        pltpu.SemaphoreType.DMA,
        pltpu.SemaphoreType.DMA,
    ],
)
def _gather(tab2d, idx_lin, out5, idxall, rows0, rows1, outb0, outb1,
            gsem0, gsem1, wsem0, wsem1):
    wid = lax.axis_index("s") * NC + lax.axis_index("c")
    i16 = _iota16()
    rows_ = (rows0, rows1)
    outb_ = (outb0, outb1)
    gsem_ = (gsem0, gsem1)
    wsem_ = (wsem0, wsem1)
    nbt = BATCH // 128

    base_g = wid * BLK_PER_W
    pltpu.sync_copy(idx_lin.at[pl.ds(base_g * 128, BLK_PER_W * 128)], idxall)

    def g_desc(j, p):
        return pltpu.make_async_copy(
            tab2d.at[idxall.at[pl.ds(j * CHUNK_BLKS * 128, CHUNK_BLKS * 128)]],
            rows_[p], gsem_[p])

    half = CHUNK_BLKS * 8 * 128

    def out_descs(j, p):
        g0 = base_g + CHUNK_BLKS * j
        f = g0 // nbt
        bt0 = g0 % nbt
        return (
            pltpu.make_async_copy(
                outb_[p].at[pl.ds(0, half)],
                out5.at[f, 0, pl.ds(bt0 * 1024, half)], wsem_[p]),
            pltpu.make_async_copy(
                outb_[p].at[pl.ds(half, half)],
                out5.at[f, 1, pl.ds(bt0 * 1024, half)], wsem_[p]),
        )

    diag = [(i16 + d) & 15 for d in range(16)]
    dstb = [(diag[d] // 8) * half + (diag[d] % 8) * 128 + i16
            for d in range(16)]

    def transpose_chunk(p):
        def m_body(m, c2, p=p):
            rv = m * 16 + i16
            dbase = (m // 8) * 1024 + (m % 8) * 16
            for d in range(16):
                v = plsc.load_gather(rows_[p], [rv, diag[d]])
                plsc.store_scatter(outb_[p], [dstb[d] + dbase], v)
            return c2

        lax.fori_loop(0, CHUNK_BLKS * 8, m_body, 0)

    def do_chunk(j, p):
        transpose_chunk(p)
        d1, d2 = out_descs(j, p)
        d1.start()
        d2.start()

    def wait_outs(j, p):
        d1, d2 = out_descs(j, p)
        d1.wait()
        d2.wait()

    g_desc(0, 0).start()
    g_desc(1, 1).start()
    g_desc(0, 0).wait()
    do_chunk(0, 0)
    g_desc(2, 0).start()
    g_desc(1, 1).wait()
    do_chunk(1, 1)
    g_desc(3, 1).start()

    def pair_body(u, c):
        j0 = 2 * u
        j1 = j0 + 1
        g_desc(j0, 0).wait()
        wait_outs(j0 - 2, 0)
        do_chunk(j0, 0)
        g_desc(j0 + 2, 0).start()
        g_desc(j1, 1).wait()
        wait_outs(j1 - 2, 1)
        do_chunk(j1, 1)

        @pl.when(j1 + 2 < NCHUNK)
        def _():
            g_desc(j1 + 2, 1).start()

        return c

    lax.fori_loop(1, (NCHUNK - 1) // 2, pair_body, 0)
    g_desc(NCHUNK - 1, 0).wait()
    wait_outs(NCHUNK - 3, 0)
    do_chunk(NCHUNK - 1, 0)
    wait_outs(NCHUNK - 2, 1)
    wait_outs(NCHUNK - 1, 0)


@jax.jit
def kernel(indices, table):
    tab_lin, idx_lin = _detile(table.T, indices.T)
    out5 = _gather(tab_lin.reshape(VOCAB, EMBED_DIM), idx_lin)
    out5 = out5.reshape(N_FIELDS, 2, BATCH // 128, 8, 128)
    return out5.transpose(2, 4, 0, 1, 3).reshape(BATCH, N_FIELDS, EMBED_DIM)


# speedup vs baseline: 1.4691x; 1.0036x over previous
import functools

import jax
import jax.numpy as jnp
from jax import lax
from jax.experimental import pallas as pl
from jax.experimental.pallas import tpu as pltpu
from jax.experimental.pallas import tpu_sc as plsc

BATCH = 16384
N_FIELDS = 26
EMBED_DIM = 16
VOCAB = 1000000

B = BATCH * N_FIELDS
NC, NS = 2, 16
NW = NC * NS
C_SB = 512
N_SB_W = 61
SB_COVER = NW * N_SB_W * C_SB
N_LEFT = (VOCAB - SB_COVER) // 128
TAIL_START = SB_COVER + N_LEFT * 128
TAIL_N = VOCAB - TAIL_START
NBLOCKS = N_FIELDS * (BATCH // 128)
BLK_PER_W = NBLOCKS // NW
CHUNK_BLKS = 8
NCHUNK = BLK_PER_W // CHUNK_BLKS

_mesh = plsc.VectorSubcoreMesh(core_axis_name="c", subcore_axis_name="s")

_detile_params = pltpu.CompilerParams(use_tc_tiling_on_sc=True,
                                      needs_layout_passes=False)
_gather_params = pltpu.CompilerParams(use_tc_tiling_on_sc=False,
                                      needs_layout_passes=False)


def _iota16():
    return lax.iota(jnp.int32, 16)


@functools.partial(
    pl.kernel,
    mesh=_mesh,
    out_type=(
        jax.ShapeDtypeStruct((VOCAB * EMBED_DIM,), jnp.float32),
        jax.ShapeDtypeStruct((B,), jnp.int32),
    ),
    compiler_params=_detile_params,
    scratch_types=[
        pltpu.VMEM((EMBED_DIM, C_SB), jnp.float32),
        pltpu.VMEM((EMBED_DIM, C_SB), jnp.float32),
        pltpu.VMEM((C_SB * EMBED_DIM,), jnp.float32),
        pltpu.VMEM((C_SB * EMBED_DIM,), jnp.float32),
        pltpu.VMEM((BATCH,), jnp.int32),
        pltpu.VMEM((EMBED_DIM, TAIL_N), jnp.float32),
        pltpu.VMEM((TAIL_N * EMBED_DIM,), jnp.float32),
        pltpu.SemaphoreType.DMA,
        pltpu.SemaphoreType.DMA,
        pltpu.SemaphoreType.DMA,
        pltpu.SemaphoreType.DMA,
    ],
)
def _detile(tt, it, tab_lin, idx_lin, vin0, vin1, vout0, vout1, idxrow,
            tailbuf, tailout, isem0, isem1, osem0, osem1):
    wid = lax.axis_index("s") * NC + lax.axis_index("c")
    i16 = _iota16()
    vin_ = (vin0, vin1)
    vout_ = (vout0, vout1)
    isem_ = (isem0, isem1)
    osem_ = (osem0, osem1)

    @pl.when(wid < N_FIELDS)
    def _():
        pltpu.sync_copy(it.at[wid], idxrow)
        pltpu.sync_copy(idxrow, idx_lin.at[pl.ds(wid * BATCH, BATCH)])

    base_col = wid * N_SB_W * C_SB

    def in_desc(t, p):
        return pltpu.make_async_copy(
            tt.at[:, pl.ds(base_col + t * C_SB, C_SB)], vin_[p], isem_[p])

    def out_desc(t, p):
        return pltpu.make_async_copy(
            vout_[p],
            tab_lin.at[pl.ds((base_col + t * C_SB) * EMBED_DIM,
                             C_SB * EMBED_DIM)],
            osem_[p])

    diag = [(i16 + d) & 15 for d in range(16)]

    def transpose_sb(p):
        def c_body(c0w, carry):
            c0 = c0w * 16
            for d in range(16):
                cv = diag[d] + c0
                v = plsc.load_gather(vin_[p], [i16, cv])
                plsc.store_scatter(vout_[p], [cv * 16 + i16], v)
            return carry

        lax.fori_loop(0, C_SB // 16, c_body, 0)

    in_desc(0, 0).start()
    in_desc(0, 0).wait()
    in_desc(1, 1).start()
    transpose_sb(0)
    out_desc(0, 0).start()
    in_desc(2, 0).start()
    in_desc(1, 1).wait()
    transpose_sb(1)
    out_desc(1, 1).start()
    in_desc(3, 1).start()

    def pair_body(u, c):
        t0 = 2 * u
        t1 = t0 + 1
        in_desc(t0, 0).wait()
        out_desc(t0 - 2, 0).wait()
        transpose_sb(0)
        out_desc(t0, 0).start()
        in_desc(t0 + 2, 0).start()
        in_desc(t1, 1).wait()
        out_desc(t1 - 2, 1).wait()
        transpose_sb(1)
        out_desc(t1, 1).start()

        @pl.when(t1 + 2 < N_SB_W)
        def _():
            in_desc(t1 + 2, 1).start()

        return c

    lax.fori_loop(1, (N_SB_W - 1) // 2, pair_body, 0)
    in_desc(N_SB_W - 1, 0).wait()
    out_desc(N_SB_W - 3, 0).wait()
    transpose_sb(0)
    out_desc(N_SB_W - 1, 0).start()
    out_desc(N_SB_W - 2, 1).wait()
    out_desc(N_SB_W - 1, 0).wait()

    @pl.when((wid >= 16) & (wid < 16 + N_LEFT))
    def _():
        c0 = SB_COVER + (wid - 16) * 128
        pltpu.sync_copy(tt.at[:, pl.ds(c0, 128)], vin_[0].at[:, pl.ds(0, 128)])

        def trl_body(m, c2):
            for jj in range(8):
                r = m * 8 + jj
                v = plsc.load_gather(vin_[0],
                                     [i16, jnp.full((16,), r, jnp.int32)])
                vout_[0][pl.ds(r * EMBED_DIM, EMBED_DIM)] = v
            return c2

        lax.fori_loop(0, 16, trl_body, 0)
        pltpu.sync_copy(vout_[0].at[pl.ds(0, 128 * EMBED_DIM)],
                        tab_lin.at[pl.ds(c0 * EMBED_DIM, 128 * EMBED_DIM)])

    @pl.when(wid == 20)
    def _():
        def te_body(e, c2):
            pltpu.sync_copy(tt.at[e, pl.ds(TAIL_START, TAIL_N)], tailbuf.at[e])
            return c2

        lax.fori_loop(0, EMBED_DIM, te_body, 0)

        def tr2_body(r, c2):
            v = plsc.load_gather(tailbuf, [i16, jnp.full((16,), r, jnp.int32)])
            tailout[pl.ds(r * EMBED_DIM, EMBED_DIM)] = v
            return c2

        lax.fori_loop(0, TAIL_N, tr2_body, 0)
        pltpu.sync_copy(tailout,
                        tab_lin.at[pl.ds(TAIL_START * EMBED_DIM,
                                         TAIL_N * EMBED_DIM)])


@functools.partial(
    pl.kernel,
    mesh=_mesh,
    out_type=jax.ShapeDtypeStruct((N_FIELDS, 2, BATCH * 8), jnp.float32),
    compiler_params=_gather_params,
    scratch_types=[
        pltpu.VMEM((BLK_PER_W * 128,), jnp.int32),
        pltpu.VMEM((CHUNK_BLKS * 128, EMBED_DIM), jnp.float32),
        pltpu.VMEM((CHUNK_BLKS * 128, EMBED_DIM), jnp.float32),
        pltpu.VMEM((2 * CHUNK_BLKS * 8 * 128,), jnp.float32),
        pltpu.VMEM((2 * CHUNK_BLKS * 8 * 128,), jnp.float32),
        pltpu.SemaphoreType.DMA,
        pltpu.SemaphoreType.DMA,
        pltpu.SemaphoreType.DMA,
        pltpu.SemaphoreType.DMA,
    ],
)
def _gather(tab2d, idx_lin, out5, idxall, rows0, rows1, outb0, outb1,
            gsem0, gsem1, wsem0, wsem1):
    wid = lax.axis_index("s") * NC + lax.axis_index("c")
    i16 = _iota16()
    rows_ = (rows0, rows1)
    outb_ = (outb0, outb1)
    gsem_ = (gsem0, gsem1)
    wsem_ = (wsem0, wsem1)
    nbt = BATCH // 128

    base_g = wid * BLK_PER_W
    pltpu.sync_copy(idx_lin.at[pl.ds(base_g * 128, BLK_PER_W * 128)], idxall)

    def g_desc(j, p):
        return pltpu.make_async_copy(
            tab2d.at[idxall.at[pl.ds(j * CHUNK_BLKS * 128, CHUNK_BLKS * 128)]],
            rows_[p], gsem_[p])

    half = CHUNK_BLKS * 8 * 128

    def out_descs(j, p):
        g0 = base_g + CHUNK_BLKS * j
        f = g0 // nbt
        bt0 = g0 % nbt
        return (
            pltpu.make_async_copy(
                outb_[p].at[pl.ds(0, half)],
                out5.at[f, 0, pl.ds(bt0 * 1024, half)], wsem_[p]),
            pltpu.make_async_copy(
                outb_[p].at[pl.ds(half, half)],
                out5.at[f, 1, pl.ds(bt0 * 1024, half)], wsem_[p]),
        )

    diag = [(i16 + d) & 15 for d in range(16)]
    dstb = [(diag[d] // 8) * half + (diag[d] % 8) * 128 + i16
            for d in range(16)]

    def transpose_chunk(p):
        def m_body(m, c2, p=p):
            rv = m * 16 + i16
            dbase = (m // 8) * 1024 + (m % 8) * 16
            for d in range(16):
                v = plsc.load_gather(rows_[p], [rv, diag[d]])
                plsc.store_scatter(outb_[p], [dstb[d] + dbase], v)
            return c2

        lax.fori_loop(0, CHUNK_BLKS * 8, m_body, 0)

    def do_chunk(j, p):
        transpose_chunk(p)
        d1, d2 = out_descs(j, p)
        d1.start()
        d2.start()

    def wait_outs(j, p):
        d1, d2 = out_descs(j, p)
        d1.wait()
        d2.wait()

    g_desc(0, 0).start()
    g_desc(1, 1).start()
    g_desc(0, 0).wait()
    do_chunk(0, 0)
    g_desc(2, 0).start()
    g_desc(1, 1).wait()
    do_chunk(1, 1)
    g_desc(3, 1).start()

    def pair_body(u, c):
        j0 = 2 * u
        j1 = j0 + 1
        g_desc(j0, 0).wait()
        wait_outs(j0 - 2, 0)
        do_chunk(j0, 0)
        g_desc(j0 + 2, 0).start()
        g_desc(j1, 1).wait()
        wait_outs(j1 - 2, 1)
        do_chunk(j1, 1)

        @pl.when(j1 + 2 < NCHUNK)
        def _():
            g_desc(j1 + 2, 1).start()

        return c

    lax.fori_loop(1, (NCHUNK - 1) // 2, pair_body, 0)
    g_desc(NCHUNK - 1, 0).wait()
    wait_outs(NCHUNK - 3, 0)
    do_chunk(NCHUNK - 1, 0)
    wait_outs(NCHUNK - 2, 1)
    wait_outs(NCHUNK - 1, 0)


@jax.jit
def kernel(indices, table):
    tab_lin, idx_lin = _detile(table.T, indices.T)
    out5 = _gather(tab_lin.reshape(VOCAB, EMBED_DIM), idx_lin)
    out5 = out5.reshape(N_FIELDS, 2, BATCH // 128, 8, 128)
    return out5.transpose(2, 4, 0, 1, 3).reshape(BATCH, N_FIELDS, EMBED_DIM)
